# baseline (speedup 1.0000x reference)
import functools

import jax
import jax.numpy as jnp
from jax import lax
from jax.experimental import pallas as pl
from jax.experimental.pallas import tpu as pltpu
from jax.experimental.pallas import tpu_sc as plsc

NUM_TYPES = 7
F = 32
NC = 2
NS = 16
LANES = 16


def _make_seg_sum(n_nodes: int, n_edges: int, chunk: int, unroll: int = 8,
                  pool: bool = False):
    assert n_nodes % LANES == 0 and chunk % LANES == 0 and n_edges % chunk == 0
    n_chunks = n_edges // chunk
    out_cols = LANES if pool else n_nodes
    mesh = plsc.VectorSubcoreMesh(core_axis_name="c", subcore_axis_name="s")

    @functools.partial(
        pl.kernel,
        mesh=mesh,
        out_type=jax.ShapeDtypeStruct((F, out_cols), jnp.float32),
        compiler_params=pltpu.CompilerParams(needs_layout_passes=False),
        scratch_types=[
            pltpu.VMEM((n_nodes,), jnp.float32),
            pltpu.VMEM((n_nodes,), jnp.float32),
            pltpu.VMEM((chunk,), jnp.int32),
            pltpu.VMEM((chunk,), jnp.int32),
            pltpu.VMEM((chunk,), jnp.float32),
            pltpu.VMEM((chunk,), jnp.float32),
            pltpu.VMEM((LANES,), jnp.float32),
            pltpu.SemaphoreType.DMA,
            pltpu.SemaphoreType.DMA,
        ],
    )
    def seg(q_hbm, pk_hbm, w_hbm, bias_hbm, out_hbm,
            q_v, m_v, pk0, pk1, w0, w1, b_v, sem0, sem1):
        wid = lax.axis_index("s") * NC + lax.axis_index("c")
        pk_bufs = (pk0, pk1)
        w_bufs = (w0, w1)
        sems = (sem0, sem1)

        def issue(c):
            b = c % 2
            base = c * chunk
            return (
                pltpu.async_copy(pk_hbm.at[pl.ds(base, chunk)], pk_bufs[b], sems[b]),
                pltpu.async_copy(w_hbm.at[pl.ds(base, chunk)], w_bufs[b], sems[b]),
            )

        descs = [None, None]
        descs[0] = issue(0)
        if n_chunks > 1:
            descs[1] = issue(1)

        pltpu.sync_copy(q_hbm.at[wid], q_v)
        pltpu.sync_copy(bias_hbm.at[wid], b_v)
        bval = b_v[...]

        @plsc.parallel_loop(0, n_nodes, LANES, unroll=unroll)
        def _(i):
            m_v[pl.ds(i, LANES)] = bval

        for c in range(n_chunks):
            b = c % 2
            for d in descs[b]:
                d.wait()
            pk_v, w_v = pk_bufs[b], w_bufs[b]

            @plsc.parallel_loop(0, chunk, LANES, unroll=unroll)
            def _(j):
                sl = pl.ds(j, LANES)
                p = pk_v[sl]
                s = lax.shift_right_logical(p, 16)
                d_idx = lax.bitwise_and(p, 0xFFFF)
                vals = plsc.load_gather(q_v, [s])
                plsc.addupdate_scatter(m_v, [d_idx], vals * w_v[sl])

            if c + 2 < n_chunks:
                descs[b] = issue(c + 2)

        if pool:
            one = jnp.ones((LANES,), jnp.float32)

            @plsc.parallel_loop(0, n_nodes, LANES, carry=jnp.zeros((LANES,), jnp.float32))
            def acc_loop(i, acc):
                mv = m_v[pl.ds(i, LANES)]
                return acc + one / (one + jnp.exp(-mv))

            b_v[...] = acc_loop
            pltpu.sync_copy(b_v, out_hbm.at[wid])
        else:
            pltpu.sync_copy(m_v, out_hbm.at[wid])

    return seg


def _make_typed_matmul(din: int, n_nodes: int, leaky: bool, n_edges: int = 0):
    def body(*refs):
        if n_edges:
            x_ref, t_ref, v_ref, e_ref, out_ref, pk_ref = refs
            pk_ref[...] = (e_ref[0, :][None] << 16) | e_ref[1, :][None]
        else:
            x_ref, t_ref, v_ref, out_ref = refs
        x = x_ref[...]
        if leaky:
            x = jnp.where(x >= 0, x, 0.01 * x)
        t = t_ref[0]
        acc = jnp.zeros((F, n_nodes), jnp.float32)
        for tid in range(NUM_TYPES):
            qt = lax.dot_general(
                v_ref[tid], x, (((1,), (0,)), ((), ())),
                preferred_element_type=jnp.float32,
                precision=lax.Precision.DEFAULT)
            acc = jnp.where(t == tid, qt, acc)
        out_ref[...] = acc

    in_specs = [
        pl.BlockSpec((din, n_nodes), lambda: (0, 0)),
        pl.BlockSpec((1, 1, n_nodes), lambda: (0, 0, 0)),
        pl.BlockSpec((NUM_TYPES, F, din), lambda: (0, 0, 0)),
    ]
    out_specs = pl.BlockSpec((F, n_nodes), lambda: (0, 0))
    out_shape = jax.ShapeDtypeStruct((F, n_nodes), jnp.float32)
    if n_edges:
        in_specs.append(pl.BlockSpec((2, n_edges), lambda: (0, 0)))
        out_specs = [out_specs, pl.BlockSpec((1, n_edges), lambda: (0, 0))]
        out_shape = [out_shape, jax.ShapeDtypeStruct((1, n_edges), jnp.int32)]
    return pl.pallas_call(
        body, in_specs=in_specs, out_specs=out_specs, out_shape=out_shape)


def kernel(x_node_feature, x_edge_index, x_edge_weight, x_node_types,
           W1, b1, U1, c1, W2, b2, U2, c2):
    n_nodes, in_dim = x_node_feature.shape
    n_edges = x_edge_index.shape[1]
    hidden = W1.shape[2]

    edge_idx = x_edge_index.astype(jnp.int32)
    w = x_edge_weight.astype(jnp.float32)
    types3 = x_node_types.astype(jnp.int32).reshape(1, 1, n_nodes)

    U1r = U1.reshape(NUM_TYPES, hidden, F)
    V1T = jnp.einsum('tih,tho->toi', W1, U1r)
    const1 = b1.reshape(-1) @ U1 + c1
    bias1 = jnp.broadcast_to(const1[:, None], (F, LANES))
    U2r = U2.reshape(NUM_TYPES, hidden, F)
    V2T = jnp.einsum('tih,tho->toi', W2, U2r)
    const2 = b2.reshape(-1) @ U2 + c2
    bias2 = jnp.broadcast_to(const2[:, None], (F, LANES))

    xT = x_node_feature.T
    seg = _make_seg_sum(n_nodes, n_edges, chunk=20000)
    seg_pool = _make_seg_sum(n_nodes, n_edges, chunk=20000, pool=True)
    q1T, packed2 = _make_typed_matmul(in_dim, n_nodes, leaky=False,
                                      n_edges=n_edges)(xT, types3, V1T, edge_idx)
    packed = packed2.reshape(n_edges)
    m1T = seg(q1T, packed, w, bias1)
    q2T = _make_typed_matmul(F, n_nodes, leaky=True)(m1T, types3, V2T)
    pooled = seg_pool(q2T, packed, w, bias2)
    return jnp.sum(pooled, axis=1)


# speedup vs baseline: 1.0240x; 1.0240x over previous
import functools

import jax
import jax.numpy as jnp
from jax import lax
from jax.experimental import pallas as pl
from jax.experimental.pallas import tpu as pltpu
from jax.experimental.pallas import tpu_sc as plsc

NUM_TYPES = 7
F = 32
NC = 2
NS = 16
LANES = 16


def _make_seg_sum(n_nodes: int, n_edges: int, chunk: int, unroll: int = 8,
                  pool: bool = False):
    assert n_nodes % LANES == 0 and chunk % LANES == 0 and n_edges % chunk == 0
    n_chunks = n_edges // chunk
    out_cols = LANES if pool else n_nodes
    mesh = plsc.VectorSubcoreMesh(core_axis_name="c", subcore_axis_name="s")

    @functools.partial(
        pl.kernel,
        mesh=mesh,
        out_type=jax.ShapeDtypeStruct((F, out_cols), jnp.float32),
        compiler_params=pltpu.CompilerParams(needs_layout_passes=False),
        scratch_types=[
            pltpu.VMEM((n_nodes,), jnp.float32),
            pltpu.VMEM((n_nodes,), jnp.float32),
            pltpu.VMEM((chunk,), jnp.int32),
            pltpu.VMEM((chunk,), jnp.int32),
            pltpu.VMEM((chunk,), jnp.float32),
            pltpu.VMEM((chunk,), jnp.float32),
            pltpu.VMEM((LANES,), jnp.float32),
            pltpu.SemaphoreType.DMA,
            pltpu.SemaphoreType.DMA,
        ],
    )
    def seg(q_hbm, pk_hbm, w_hbm, bias_hbm, out_hbm,
            q_v, m_v, pk0, pk1, w0, w1, b_v, sem0, sem1):
        wid = lax.axis_index("s") * NC + lax.axis_index("c")
        pk_bufs = (pk0, pk1)
        w_bufs = (w0, w1)
        sems = (sem0, sem1)

        def issue(c):
            b = c % 2
            base = c * chunk
            return (
                pltpu.async_copy(pk_hbm.at[pl.ds(base, chunk)], pk_bufs[b], sems[b]),
                pltpu.async_copy(w_hbm.at[pl.ds(base, chunk)], w_bufs[b], sems[b]),
            )

        descs = [None, None]
        descs[0] = issue(0)
        if n_chunks > 1:
            descs[1] = issue(1)

        pltpu.sync_copy(q_hbm.at[wid], q_v)
        pltpu.sync_copy(bias_hbm.at[wid], b_v)
        bval = b_v[...]

        @plsc.parallel_loop(0, n_nodes, LANES, unroll=unroll)
        def _(i):
            m_v[pl.ds(i, LANES)] = bval

        for c in range(n_chunks):
            b = c % 2
            for d in descs[b]:
                d.wait()
            pk_v, w_v = pk_bufs[b], w_bufs[b]

            @plsc.parallel_loop(0, chunk, LANES, unroll=unroll)
            def _(j):
                sl = pl.ds(j, LANES)
                p = pk_v[sl]
                s = lax.shift_right_logical(p, 16)
                d_idx = lax.bitwise_and(p, 0xFFFF)
                vals = plsc.load_gather(q_v, [s])
                plsc.addupdate_scatter(m_v, [d_idx], vals * w_v[sl])

            if c + 2 < n_chunks:
                descs[b] = issue(c + 2)

        if pool:
            one = jnp.ones((LANES,), jnp.float32)

            @plsc.parallel_loop(0, n_nodes, LANES, carry=jnp.zeros((LANES,), jnp.float32))
            def acc_loop(i, acc):
                mv = m_v[pl.ds(i, LANES)]
                return acc + one / (one + jnp.exp(-mv))

            b_v[...] = acc_loop
            pltpu.sync_copy(b_v, out_hbm.at[wid])
        else:
            pltpu.sync_copy(m_v, out_hbm.at[wid])

    return seg


def _make_typed_matmul(din: int, n_nodes: int, leaky: bool, n_edges: int = 0):
    def body(*refs):
        if n_edges:
            x_ref, t_ref, v_ref, e_ref, out_ref, pk_ref = refs
            pk_ref[...] = (e_ref[0, :][None] << 16) | e_ref[1, :][None]
        else:
            x_ref, t_ref, v_ref, out_ref = refs
        x = x_ref[...]
        if leaky:
            x = jnp.where(x >= 0, x, 0.01 * x)
        t = t_ref[0]
        acc = jnp.zeros((F, n_nodes), jnp.float32)
        for tid in range(NUM_TYPES):
            qt = lax.dot_general(
                v_ref[tid], x, (((1,), (0,)), ((), ())),
                preferred_element_type=jnp.float32,
                precision=lax.Precision.DEFAULT)
            acc = jnp.where(t == tid, qt, acc)
        out_ref[...] = acc

    in_specs = [
        pl.BlockSpec((din, n_nodes), lambda: (0, 0)),
        pl.BlockSpec((1, 1, n_nodes), lambda: (0, 0, 0)),
        pl.BlockSpec((NUM_TYPES, F, din), lambda: (0, 0, 0)),
    ]
    out_specs = pl.BlockSpec((F, n_nodes), lambda: (0, 0))
    out_shape = jax.ShapeDtypeStruct((F, n_nodes), jnp.float32)
    if n_edges:
        in_specs.append(pl.BlockSpec((2, n_edges), lambda: (0, 0)))
        out_specs = [out_specs, pl.BlockSpec((1, n_edges), lambda: (0, 0))]
        out_shape = [out_shape, jax.ShapeDtypeStruct((1, n_edges), jnp.int32)]
    return pl.pallas_call(
        body, in_specs=in_specs, out_specs=out_specs, out_shape=out_shape)


def kernel(x_node_feature, x_edge_index, x_edge_weight, x_node_types,
           W1, b1, U1, c1, W2, b2, U2, c2):
    n_nodes, in_dim = x_node_feature.shape
    n_edges = x_edge_index.shape[1]
    hidden = W1.shape[2]

    edge_idx = x_edge_index.astype(jnp.int32)
    w = x_edge_weight.astype(jnp.float32)
    types3 = x_node_types.astype(jnp.int32).reshape(1, 1, n_nodes)

    U1r = U1.reshape(NUM_TYPES, hidden, F)
    V1T = jnp.einsum('tih,tho->toi', W1, U1r)
    const1 = b1.reshape(-1) @ U1 + c1
    bias1 = jnp.broadcast_to(const1[:, None], (F, LANES))
    U2r = U2.reshape(NUM_TYPES, hidden, F)
    V2T = jnp.einsum('tih,tho->toi', W2, U2r)
    const2 = b2.reshape(-1) @ U2 + c2
    bias2 = jnp.broadcast_to(const2[:, None], (F, LANES))

    xT = x_node_feature.T
    seg = _make_seg_sum(n_nodes, n_edges, chunk=16000)
    seg_pool = _make_seg_sum(n_nodes, n_edges, chunk=16000, pool=True)
    q1T, packed2 = _make_typed_matmul(in_dim, n_nodes, leaky=False,
                                      n_edges=n_edges)(xT, types3, V1T, edge_idx)
    packed = packed2.reshape(n_edges)
    m1T = seg(q1T, packed, w, bias1)
    q2T = _make_typed_matmul(F, n_nodes, leaky=True)(m1T, types3, V2T)
    pooled = seg_pool(q2T, packed, w, bias2)
    return jnp.sum(pooled, axis=1)
